# baseline (speedup 1.0000x reference)
import functools

import jax
import jax.numpy as jnp
from jax import lax
from jax.experimental import pallas as pl
from jax.experimental.pallas import tpu as pltpu
from jax.experimental.pallas import tpu_sc as plsc

_L = 16
_D = 128
_D_IN = 260
_N = 100000
_CHUNK = 125
_BUFROWS = 136
_NW = 32
_PER_W = 25
_WSTRIDE = 3200
_NG = 8


def _sc_scores_body(hv_hbm, w1a_hbm, w2_hbm, out_hbm,
                    buf, w1buf, w2buf, vabuf, sbuf):
    wid = lax.axis_index("s") * 2 + lax.axis_index("c")
    iota = lax.iota(jnp.int32, _L)

    pltpu.sync_copy(w1a_hbm, w1buf)
    pltpu.sync_copy(w2_hbm, w2buf)
    w1rows = [(16 * j + iota) * _D_IN for j in range(_D // _L)]

    def va_step(k, accs):
        kv = jnp.full((_L,), k, jnp.int32)
        w2k = plsc.load_gather(w2buf, [kv])
        return tuple(
            accs[j] + plsc.load_gather(w1buf, [w1rows[j] + kv]) * w2k
            for j in range(_D // _L))

    va = lax.fori_loop(0, _D_IN, va_step,
                       tuple(jnp.zeros((_L,), jnp.float32)
                             for _ in range(_D // _L)))
    for j in range(_D // _L):
        vabuf[pl.ds(16 * j, _L)] = va[j]

    def chunk_step(t, carry):
        start = (wid * _PER_W + t) * _CHUNK
        aligned = pl.multiple_of(
            jnp.minimum((start // 8) * 8, _N - _BUFROWS), 8)
        extra = start - aligned
        pltpu.sync_copy(hv_hbm.at[pl.ds(aligned, _BUFROWS)], buf)
        rows = [jnp.minimum(16 * g + iota, _CHUNK - 1) + extra
                for g in range(_NG)]

        def d_step(d, accs):
            dv = jnp.full((_L,), d, jnp.int32)
            vad = plsc.load_gather(vabuf, [dv])
            return tuple(
                accs[g] + plsc.load_gather(buf, [rows[g], dv]) * vad
                for g in range(_NG))

        accs = lax.fori_loop(
            0, _D, d_step,
            tuple(jnp.zeros((_L,), jnp.float32) for _ in range(_NG)),
            unroll=4)
        for g in range(_NG):
            sbuf[pl.ds(t * _CHUNK + 16 * g, _L)] = accs[g]
        return carry

    lax.fori_loop(0, _PER_W, chunk_step, 0)
    pltpu.sync_copy(sbuf, out_hbm.at[pl.ds(wid * _WSTRIDE, _WSTRIDE)])


def _sc_scores(hv, w1af, w2f):
    mesh = plsc.VectorSubcoreMesh(core_axis_name="c", subcore_axis_name="s")
    f = functools.partial(
        pl.kernel, mesh=mesh,
        out_type=jax.ShapeDtypeStruct((_NW * _WSTRIDE,), jnp.float32),
        scratch_types=[
            pltpu.VMEM((_BUFROWS, _D), jnp.float32),
            pltpu.VMEM((_D * _D_IN,), jnp.float32),
            pltpu.VMEM((_D_IN,), jnp.float32),
            pltpu.VMEM((_D,), jnp.float32),
            pltpu.VMEM((_WSTRIDE,), jnp.float32),
        ],
        compiler_params=pltpu.CompilerParams(needs_layout_passes=False),
    )(_sc_scores_body)
    return f(hv, w1af, w2f)


def _softmax_body(s_ref, dest_ref, probs_ref, logp_ref):
    n_per_w = _PER_W * _CHUNK
    s = s_ref[...]
    r = lax.broadcasted_iota(jnp.int32, s.shape, 0)
    c = lax.broadcasted_iota(jnp.int32, s.shape, 1)
    flat = r * n_per_w + c
    valid = (c < n_per_w) & (flat < _N - 1)
    sm = jnp.where(valid, s, jnp.float32(-1e30))
    m = jnp.max(sm)
    e = jnp.where(valid, jnp.exp(sm - m), jnp.float32(0.0))
    tot = jnp.sum(e)
    probs_ref[...] = e / tot
    sd = jnp.sum(jnp.where(valid & (flat == dest_ref[0]), sm,
                           jnp.float32(0.0)))
    logp_ref[...] = jnp.reshape(sd - m - jnp.log(tot), (1, 1))


def kernel(hv, W1, b1, W2, b2, bond_type, dest):
    n, d = hv.shape
    del b1, b2, bond_type

    w1af = W1[:d, :].reshape(-1)
    w2f = W2.reshape(-1)
    scores = _sc_scores(hv, w1af, w2f).reshape(_NW, _WSTRIDE)

    dest_arr = jnp.asarray(dest, jnp.int32).reshape(1)
    probs2d, logp = pl.pallas_call(
        _softmax_body,
        in_specs=[
            pl.BlockSpec((_NW, _WSTRIDE), lambda: (0, 0)),
            pl.BlockSpec(memory_space=pltpu.SMEM),
        ],
        out_specs=[
            pl.BlockSpec((_NW, _WSTRIDE), lambda: (0, 0)),
            pl.BlockSpec((1, 1), lambda: (0, 0)),
        ],
        out_shape=[
            jax.ShapeDtypeStruct((_NW, _WSTRIDE), jnp.float32),
            jax.ShapeDtypeStruct((1, 1), jnp.float32),
        ],
    )(scores, dest_arr)

    probs = probs2d[:, : _PER_W * _CHUNK].reshape(1, n)[:, : n - 1]
    return probs, logp


# speedup vs baseline: 4.3640x; 4.3640x over previous
import functools

import jax
import jax.numpy as jnp
from jax import lax
from jax.experimental import pallas as pl
from jax.experimental.pallas import tpu as pltpu
from jax.experimental.pallas import tpu_sc as plsc

_L = 16
_D = 128
_D_IN = 260
_N = 100000
_CHUNK = 125
_BUFROWS = 136
_NW = 32
_PER_W = 25
_WSTRIDE = 3200
_NG = 8


def _sc_scores_body(hv_hbm, w1a_hbm, w2d_hbm, out_hbm,
                    buf0, buf1, w1buf, w2buf, va2buf, sbuf,
                    semw, sem0, sem1):
    wid = lax.axis_index("s") * 2 + lax.axis_index("c")
    iota = lax.iota(jnp.int32, _L)
    first = wid * _PER_W

    def chunk_src(t):
        start = (first + t) * _CHUNK
        aligned = pl.multiple_of(
            jnp.minimum((start // 8) * 8, _N - _BUFROWS), 8)
        return hv_hbm.at[pl.ds(aligned, _BUFROWS)], start - aligned

    def fire(t, buf, sem):
        src, _ = chunk_src(t)
        pltpu.async_copy(src, buf, sem)

    pltpu.async_copy(w1a_hbm, w1buf, semw)
    fire(0, buf0, sem0)
    pltpu.sync_copy(w2d_hbm, w2buf)
    pltpu.make_async_copy(w1a_hbm, w1buf, semw).wait()

    w1base = [(16 * j + iota) * _D_IN for j in range(_D // _L)]

    def va_step(k, accs):
        kv = k + iota
        kv = jnp.where(kv >= _D_IN, kv - _D_IN, kv)
        w2k = w2buf[pl.ds(k, _L)]
        return tuple(
            accs[j] + plsc.load_gather(w1buf, [w1base[j] + kv]) * w2k
            for j in range(_D // _L))

    va = lax.fori_loop(0, _D_IN, va_step,
                       tuple(jnp.zeros((_L,), jnp.float32)
                             for _ in range(_D // _L)), unroll=4)
    for j in range(_D // _L):
        va2buf[pl.ds(16 * j, _L)] = va[j]
        va2buf[pl.ds(_D + 16 * j, _L)] = va[j]

    def compute(t, buf):
        _, extra = chunk_src(t)
        rows = [jnp.minimum(16 * g + iota, _CHUNK - 1) + extra
                for g in range(_NG)]

        def d_step(d, accs):
            m = jnp.bitwise_and(d + iota, _D - 1)
            vad = va2buf[pl.ds(d, _L)]
            return tuple(
                accs[g] + plsc.load_gather(buf, [rows[g], m]) * vad
                for g in range(_NG))

        accs = lax.fori_loop(
            0, _D, d_step,
            tuple(jnp.zeros((_L,), jnp.float32) for _ in range(_NG)),
            unroll=4)
        for g in range(_NG):
            sbuf[pl.ds(t * _CHUNK + 16 * g, _L)] = accs[g]

    def wait(t, buf, sem):
        src, _ = chunk_src(t)
        pltpu.make_async_copy(src, buf, sem).wait()

    def pair_step(p, carry):
        t0 = 2 * p
        fire(t0 + 1, buf1, sem1)
        wait(t0, buf0, sem0)
        compute(t0, buf0)
        fire(t0 + 2, buf0, sem0)
        wait(t0 + 1, buf1, sem1)
        compute(t0 + 1, buf1)
        return carry

    lax.fori_loop(0, (_PER_W - 1) // 2, pair_step, 0)
    wait(_PER_W - 1, buf0, sem0)
    compute(_PER_W - 1, buf0)
    pltpu.sync_copy(sbuf, out_hbm.at[pl.ds(wid * _WSTRIDE, _WSTRIDE)])


def _sc_scores(hv, w1af, w2d):
    mesh = plsc.VectorSubcoreMesh(core_axis_name="c", subcore_axis_name="s")
    f = functools.partial(
        pl.kernel, mesh=mesh,
        out_type=jax.ShapeDtypeStruct((_NW * _WSTRIDE,), jnp.float32),
        scratch_types=[
            pltpu.VMEM((_BUFROWS, _D), jnp.float32),
            pltpu.VMEM((_BUFROWS, _D), jnp.float32),
            pltpu.VMEM((_D * _D_IN,), jnp.float32),
            pltpu.VMEM((2 * _D_IN,), jnp.float32),
            pltpu.VMEM((2 * _D,), jnp.float32),
            pltpu.VMEM((_WSTRIDE,), jnp.float32),
            pltpu.SemaphoreType.DMA,
            pltpu.SemaphoreType.DMA,
            pltpu.SemaphoreType.DMA,
        ],
        compiler_params=pltpu.CompilerParams(needs_layout_passes=False),
    )(_sc_scores_body)
    return f(hv, w1af, w2d)


def _softmax_body(s_ref, dest_ref, probs_ref, logp_ref):
    n_per_w = _PER_W * _CHUNK
    s = s_ref[...]
    r = lax.broadcasted_iota(jnp.int32, s.shape, 0)
    c = lax.broadcasted_iota(jnp.int32, s.shape, 1)
    flat = r * n_per_w + c
    valid = (c < n_per_w) & (flat < _N - 1)
    sm = jnp.where(valid, s, jnp.float32(-1e30))
    m = jnp.max(sm)
    e = jnp.where(valid, jnp.exp(sm - m), jnp.float32(0.0))
    tot = jnp.sum(e)
    probs_ref[...] = e / tot
    sd = jnp.sum(jnp.where(valid & (flat == dest_ref[0]), sm,
                           jnp.float32(0.0)))
    logp_ref[...] = jnp.reshape(sd - m - jnp.log(tot), (1, 1))


def kernel(hv, W1, b1, W2, b2, bond_type, dest):
    n, d = hv.shape
    del b1, b2, bond_type

    w1af = W1[:d, :].reshape(-1)
    w2f = W2.reshape(-1)
    w2d = jnp.concatenate([w2f, w2f])
    scores = _sc_scores(hv, w1af, w2d).reshape(_NW, _WSTRIDE)

    dest_arr = jnp.asarray(dest, jnp.int32).reshape(1)
    probs2d, logp = pl.pallas_call(
        _softmax_body,
        in_specs=[
            pl.BlockSpec((_NW, _WSTRIDE), lambda: (0, 0)),
            pl.BlockSpec(memory_space=pltpu.SMEM),
        ],
        out_specs=[
            pl.BlockSpec((_NW, _WSTRIDE), lambda: (0, 0)),
            pl.BlockSpec((1, 1), lambda: (0, 0)),
        ],
        out_shape=[
            jax.ShapeDtypeStruct((_NW, _WSTRIDE), jnp.float32),
            jax.ShapeDtypeStruct((1, 1), jnp.float32),
        ],
    )(scores, dest_arr)

    probs = probs2d[:, : _PER_W * _CHUNK].reshape(1, n)[:, : n - 1]
    return probs, logp


# speedup vs baseline: 4.7723x; 1.0935x over previous
import functools

import jax
import jax.numpy as jnp
from jax import lax
from jax.experimental import pallas as pl
from jax.experimental.pallas import tpu as pltpu
from jax.experimental.pallas import tpu_sc as plsc

_L = 16
_D = 128
_D_IN = 260
_N = 100000
_CHUNK = 125
_BUFROWS = 136
_NW = 32
_PER_W = 25
_WSTRIDE = 3200
_NG = 8


def _sc_scores_body(hv_hbm, w1a_hbm, w2d_hbm, out_hbm,
                    buf0, buf1, buf2, w1buf, w2buf, va2buf, sbuf,
                    semw, sem0, sem1, sem2):
    wid = lax.axis_index("s") * 2 + lax.axis_index("c")
    iota = lax.iota(jnp.int32, _L)
    first = wid * _PER_W

    def chunk_src(t):
        start = (first + t) * _CHUNK
        aligned = pl.multiple_of(
            jnp.minimum((start // 8) * 8, _N - _BUFROWS), 8)
        return hv_hbm.at[pl.ds(aligned, _BUFROWS)], start - aligned

    def fire(t, buf, sem):
        src, _ = chunk_src(t)
        pltpu.async_copy(src, buf, sem)

    pltpu.async_copy(w1a_hbm, w1buf, semw)
    fire(0, buf0, sem0)
    pltpu.sync_copy(w2d_hbm, w2buf)
    pltpu.make_async_copy(w1a_hbm, w1buf, semw).wait()

    w1base = [(16 * j + iota) * _D_IN for j in range(_D // _L)]

    def va_step(k, accs):
        kv = k + iota
        kv = jnp.where(kv >= _D_IN, kv - _D_IN, kv)
        w2k = w2buf[pl.ds(k, _L)]
        return tuple(
            accs[j] + plsc.load_gather(w1buf, [w1base[j] + kv]) * w2k
            for j in range(_D // _L))

    va = lax.fori_loop(0, _D_IN, va_step,
                       tuple(jnp.zeros((_L,), jnp.float32)
                             for _ in range(_D // _L)), unroll=4)
    for j in range(_D // _L):
        va2buf[pl.ds(16 * j, _L)] = va[j]
        va2buf[pl.ds(_D + 16 * j, _L)] = va[j]

    def compute(t, buf):
        _, extra = chunk_src(t)
        rows = [jnp.minimum(16 * g + iota, _CHUNK - 1) + extra
                for g in range(_NG)]

        def d_step(d, accs):
            m = jnp.bitwise_and(d + iota, _D - 1)
            vad = va2buf[pl.ds(d, _L)]
            return tuple(
                accs[g] + plsc.load_gather(buf, [rows[g], m]) * vad
                for g in range(_NG))

        accs = lax.fori_loop(
            0, _D, d_step,
            tuple(jnp.zeros((_L,), jnp.float32) for _ in range(_NG)),
            unroll=4)
        for g in range(_NG):
            sbuf[pl.ds(t * _CHUNK + 16 * g, _L)] = accs[g]

    def wait(t, buf, sem):
        src, _ = chunk_src(t)
        pltpu.make_async_copy(src, buf, sem).wait()

    fire(1, buf1, sem1)
    fire(2, buf2, sem2)

    def triple_step(p, carry):
        t0 = 3 * p
        for i, (b, sm) in enumerate(((buf0, sem0), (buf1, sem1),
                                     (buf2, sem2))):
            t = t0 + i
            wait(t, b, sm)
            compute(t, b)

            @pl.when(t + 3 < _PER_W)
            def _():
                fire(t + 3, b, sm)
        return carry

    lax.fori_loop(0, _PER_W // 3, triple_step, 0)
    wait(_PER_W - 1, buf0, sem0)
    compute(_PER_W - 1, buf0)
    pltpu.sync_copy(sbuf, out_hbm.at[pl.ds(wid * _WSTRIDE, _WSTRIDE)])


def _sc_scores(hv, w1af, w2d):
    mesh = plsc.VectorSubcoreMesh(core_axis_name="c", subcore_axis_name="s")
    f = functools.partial(
        pl.kernel, mesh=mesh,
        out_type=jax.ShapeDtypeStruct((_NW * _WSTRIDE,), jnp.float32),
        scratch_types=[
            pltpu.VMEM((_BUFROWS, _D), jnp.float32),
            pltpu.VMEM((_BUFROWS, _D), jnp.float32),
            pltpu.VMEM((_BUFROWS, _D), jnp.float32),
            pltpu.VMEM((_D * _D_IN,), jnp.float32),
            pltpu.VMEM((2 * _D_IN,), jnp.float32),
            pltpu.VMEM((2 * _D,), jnp.float32),
            pltpu.VMEM((_WSTRIDE,), jnp.float32),
            pltpu.SemaphoreType.DMA,
            pltpu.SemaphoreType.DMA,
            pltpu.SemaphoreType.DMA,
            pltpu.SemaphoreType.DMA,
        ],
        compiler_params=pltpu.CompilerParams(needs_layout_passes=False),
    )(_sc_scores_body)
    return f(hv, w1af, w2d)


def _softmax_body(s_ref, dest_ref, probs_ref, logp_ref):
    n_per_w = _PER_W * _CHUNK
    s = s_ref[...]
    r = lax.broadcasted_iota(jnp.int32, s.shape, 0)
    c = lax.broadcasted_iota(jnp.int32, s.shape, 1)
    flat = r * n_per_w + c
    valid = (c < n_per_w) & (flat < _N - 1)
    sm = jnp.where(valid, s, jnp.float32(-1e30))
    m = jnp.max(sm)
    e = jnp.where(valid, jnp.exp(sm - m), jnp.float32(0.0))
    tot = jnp.sum(e)
    probs_ref[...] = e / tot
    sd = jnp.sum(jnp.where(valid & (flat == dest_ref[0]), sm,
                           jnp.float32(0.0)))
    logp_ref[...] = jnp.reshape(sd - m - jnp.log(tot), (1, 1))


def kernel(hv, W1, b1, W2, b2, bond_type, dest):
    n, d = hv.shape
    del b1, b2, bond_type

    w1af = W1[:d, :].reshape(-1)
    w2f = W2.reshape(-1)
    w2d = jnp.concatenate([w2f, w2f])
    scores = _sc_scores(hv, w1af, w2d).reshape(_NW, _WSTRIDE)

    dest_arr = jnp.asarray(dest, jnp.int32).reshape(1)
    probs2d, logp = pl.pallas_call(
        _softmax_body,
        in_specs=[
            pl.BlockSpec((_NW, _WSTRIDE), lambda: (0, 0)),
            pl.BlockSpec(memory_space=pltpu.SMEM),
        ],
        out_specs=[
            pl.BlockSpec((_NW, _WSTRIDE), lambda: (0, 0)),
            pl.BlockSpec((1, 1), lambda: (0, 0)),
        ],
        out_shape=[
            jax.ShapeDtypeStruct((_NW, _WSTRIDE), jnp.float32),
            jax.ShapeDtypeStruct((1, 1), jnp.float32),
        ],
    )(scores, dest_arr)

    probs = probs2d[:, : _PER_W * _CHUNK].reshape(1, n)[:, : n - 1]
    return probs, logp
